# baseline (speedup 1.0000x reference)
import jax
import jax.numpy as jnp
from jax import lax
from jax.experimental import pallas as pl
from jax.experimental.pallas import tpu as pltpu
from jax.experimental.pallas import tpu_sc as plsc

N = 100000
F = 128
B = 256
NC = 2
NS = 16
L = 16

CH = 160
GR = 32
G = N // CH
NMAX = (G + NS - 1) // NS
HB = B // NC
ROWS_OUT = HB // NS
NFV = F // L


def _sc_body(attr_hbm, w_hbm, bi_hbm, out_hbm, chunk0, chunk1, wv0, wv1, bib,
             run_v, acc, red, tmp, shared, sem0, sem1, sem_bi):
    c = lax.axis_index("c")
    s = lax.axis_index("s")
    lo = c * HB

    zvec = jnp.zeros((L,), jnp.float32)
    n_my = (G - s + NS - 1) // NS

    def bi_start(i, carry):
        g = s + i * NS
        pltpu.async_copy(bi_hbm.at[pl.ds(g * CH, CH)],
                         bib.at[pl.ds(i * CH, CH)], sem_bi)
        return carry

    lax.fori_loop(0, n_my, bi_start, 0)

    def zrow(r, carry):
        for f in range(NFV):
            acc[lo + r, pl.ds(f * L, L)] = zvec
        return carry

    lax.fori_loop(0, HB, zrow, 0)

    def bi_drain(i, carry):
        pltpu.make_async_copy(bi_hbm.at[pl.ds(s * CH, CH)],
                              bib.at[pl.ds(i * CH, CH)], sem_bi).wait()
        return carry

    lax.fori_loop(0, n_my, bi_drain, 0)

    bufs = ((chunk0, wv0, sem0), (chunk1, wv1, sem1))

    def flags(i):
        off = i * CH
        b_lo = bib[pl.ds(off, L)][0]
        b_hi = bib[pl.ds(off + CH - L, L)][L - 1]
        return b_lo, jnp.logical_and(b_hi >= lo, b_lo < lo + HB)

    def start_dma(i, buf):
        chunk_v, wv_v, sem = buf
        r0 = (s + i * NS) * CH
        pltpu.async_copy(attr_hbm.at[pl.ds(r0, CH)], chunk_v, sem)
        pltpu.async_copy(w_hbm.at[pl.ds(r0, CH)], wv_v, sem)

    def maybe_start(i, buf):
        _, proc = flags(i)

        @pl.when(jnp.logical_and(i < n_my, proc))
        def _():
            start_dma(i, buf)

    def process_chunk(i, buf):
        chunk_v, wv_v, sem = buf
        b_lo, proc = flags(i)

        @pl.when(jnp.logical_and(i < n_my, proc))
        def _():
            r0 = (s + i * NS) * CH
            pltpu.make_async_copy(attr_hbm.at[pl.ds(r0, CH)], chunk_v,
                                  sem).wait()
            pltpu.make_async_copy(w_hbm.at[pl.ds(r0, CH)], wv_v, sem).wait()
            for f in range(NFV):
                run_v[pl.ds(f * L, L)] = zvec
            off = i * CH

            def group(r32, bp):
                rr = r32 * GR
                bvs = [bib[pl.ds(off + rr + h * L, L)] for h in range(GR // L)]
                wvs = [wv_v[pl.ds(rr + h * L, L)] for h in range(GR // L)]
                b_first = bvs[0][0]
                b_last = bvs[-1][L - 1]
                accs = [run_v[pl.ds(f * L, L)] for f in range(NFV)]

                @pl.when(b_first == b_last)
                def fast():
                    pred = b_first != bp

                    @pl.when(pred)
                    def flush():
                        for f in range(NFV):
                            plsc.addupdate(acc.at[bp, pl.ds(f * L, L)],
                                           accs[f])

                    keep = jnp.where(pred, 0.0, 1.0).astype(jnp.float32)
                    a2 = [a * keep for a in accs]
                    for h in range(GR // L):
                        for l in range(L):
                            wr = wvs[h][l]
                            for f in range(NFV):
                                a2[f] = a2[f] + chunk_v[rr + h * L + l,
                                                        pl.ds(f * L, L)] * wr
                    for f in range(NFV):
                        run_v[pl.ds(f * L, L)] = a2[f]

                @pl.when(b_first != b_last)
                def slow():
                    bpl = bp
                    a2 = list(accs)
                    for h in range(GR // L):
                        for l in range(L):
                            b = bvs[h][l]
                            wr = wvs[h][l]
                            pred = b != bpl

                            @pl.when(pred)
                            def flush(bpl=bpl, a2=list(a2)):
                                for f in range(NFV):
                                    plsc.addupdate(
                                        acc.at[bpl, pl.ds(f * L, L)], a2[f])

                            keep = jnp.where(pred, 0.0,
                                             1.0).astype(jnp.float32)
                            for f in range(NFV):
                                a2[f] = (a2[f] * keep
                                         + chunk_v[rr + h * L + l,
                                                   pl.ds(f * L, L)] * wr)
                            bpl = b
                    for f in range(NFV):
                        run_v[pl.ds(f * L, L)] = a2[f]

                return b_last

            bp_end = lax.fori_loop(0, CH // GR, group, b_lo)
            for f in range(NFV):
                sl = pl.ds(f * L, L)
                plsc.addupdate(acc.at[bp_end, sl], run_v[sl])

    maybe_start(0, bufs[0])

    def outer(o, carry):
        i0 = o * 2
        maybe_start(i0 + 1, bufs[1])
        process_chunk(i0, bufs[0])
        maybe_start(i0 + 2, bufs[0])
        process_chunk(i0 + 1, bufs[1])
        return carry

    lax.fori_loop(0, NMAX // 2, outer, 0)

    pltpu.sync_copy(acc.at[pl.ds(lo, HB)], shared.at[s])
    plsc.subcore_barrier()

    ro = s * ROWS_OUT
    pltpu.sync_copy(shared.at[0, pl.ds(ro, ROWS_OUT)], red)

    def redj(j, carry):
        pltpu.sync_copy(shared.at[j, pl.ds(ro, ROWS_OUT)], tmp)
        for r in range(ROWS_OUT):
            for f in range(NFV):
                sl = pl.ds(f * L, L)
                red[r, sl] = red[r, sl] + tmp[r, sl]
        return carry

    lax.fori_loop(1, NS, redj, 0)
    pltpu.sync_copy(red, out_hbm.at[pl.ds(lo + ro, ROWS_OUT)])


@jax.jit
def _pool(attr, w, bi):
    mesh = plsc.VectorSubcoreMesh(core_axis_name="c", subcore_axis_name="s",
                                  num_cores=NC, num_subcores=NS)
    return pl.kernel(
        _sc_body,
        out_type=jax.ShapeDtypeStruct((B, F), jnp.float32),
        mesh=mesh,
        scratch_types=[
            pltpu.VMEM((CH, F), jnp.float32),
            pltpu.VMEM((CH, F), jnp.float32),
            pltpu.VMEM((CH,), jnp.float32),
            pltpu.VMEM((CH,), jnp.float32),
            pltpu.VMEM((NMAX * CH,), jnp.int32),
            pltpu.VMEM((F,), jnp.float32),
            pltpu.VMEM((B, F), jnp.float32),
            pltpu.VMEM((ROWS_OUT, F), jnp.float32),
            pltpu.VMEM((ROWS_OUT, F), jnp.float32),
            pltpu.VMEM_SHARED((NS, HB, F), jnp.float32),
            pltpu.SemaphoreType.DMA,
            pltpu.SemaphoreType.DMA,
            pltpu.SemaphoreType.DMA,
        ],
    )(attr, w, bi)


def kernel(reference, attr, weights, batch_index):
    del reference
    return _pool(attr, weights.reshape(-1).astype(jnp.float32),
                 batch_index.astype(jnp.int32))


# speedup vs baseline: 1.1374x; 1.1374x over previous
import jax
import jax.numpy as jnp
from jax import lax
from jax.experimental import pallas as pl
from jax.experimental.pallas import tpu as pltpu
from jax.experimental.pallas import tpu_sc as plsc

N = 100000
F = 128
B = 256
NC = 2
NS = 16
L = 16

CH = 160
GR = 16
G = N // CH
NMAX = (G + NS - 1) // NS
HB = B // NC
ROWS_OUT = HB // NS
NFV = F // L


def _sc_body(attr_hbm, w_hbm, bi_hbm, out_hbm, chunk0, chunk1, wv0, wv1, bib,
             run_v, acc, red, tmp, shared, sem0, sem1, sem_bi):
    c = lax.axis_index("c")
    s = lax.axis_index("s")
    lo = c * HB

    zvec = jnp.zeros((L,), jnp.float32)
    n_my = (G - s + NS - 1) // NS

    def bi_start(i, carry):
        g = s + i * NS
        pltpu.async_copy(bi_hbm.at[pl.ds(g * CH, CH)],
                         bib.at[pl.ds(i * CH, CH)], sem_bi)
        return carry

    lax.fori_loop(0, n_my, bi_start, 0)

    def zrow(r, carry):
        for f in range(NFV):
            acc[lo + r, pl.ds(f * L, L)] = zvec
        return carry

    lax.fori_loop(0, HB, zrow, 0)

    def bi_drain(i, carry):
        pltpu.make_async_copy(bi_hbm.at[pl.ds(s * CH, CH)],
                              bib.at[pl.ds(i * CH, CH)], sem_bi).wait()
        return carry

    lax.fori_loop(0, n_my, bi_drain, 0)

    bufs = ((chunk0, wv0, sem0), (chunk1, wv1, sem1))

    def flags(i):
        off = i * CH
        b_lo = bib[pl.ds(off, L)][0]
        b_hi = bib[pl.ds(off + CH - L, L)][L - 1]
        return b_lo, jnp.logical_and(b_hi >= lo, b_lo < lo + HB)

    def start_dma(i, buf):
        chunk_v, wv_v, sem = buf
        r0 = (s + i * NS) * CH
        pltpu.async_copy(attr_hbm.at[pl.ds(r0, CH)], chunk_v, sem)
        pltpu.async_copy(w_hbm.at[pl.ds(r0, CH)], wv_v, sem)

    def maybe_start(i, buf):
        _, proc = flags(i)

        @pl.when(jnp.logical_and(i < n_my, proc))
        def _():
            start_dma(i, buf)

    def process_chunk(i, buf):
        chunk_v, wv_v, sem = buf
        b_lo, proc = flags(i)

        @pl.when(jnp.logical_and(i < n_my, proc))
        def _():
            r0 = (s + i * NS) * CH
            pltpu.make_async_copy(attr_hbm.at[pl.ds(r0, CH)], chunk_v,
                                  sem).wait()
            pltpu.make_async_copy(w_hbm.at[pl.ds(r0, CH)], wv_v, sem).wait()
            for f in range(NFV):
                run_v[pl.ds(f * L, L)] = zvec
            off = i * CH

            def group(r32, bp):
                rr = r32 * GR
                bvs = [bib[pl.ds(off + rr + h * L, L)] for h in range(GR // L)]
                wvs = [wv_v[pl.ds(rr + h * L, L)] for h in range(GR // L)]
                b_first = bvs[0][0]
                b_last = bvs[-1][L - 1]
                accs = [run_v[pl.ds(f * L, L)] for f in range(NFV)]

                @pl.when(b_first == b_last)
                def fast():
                    pred = b_first != bp

                    @pl.when(pred)
                    def flush():
                        for f in range(NFV):
                            plsc.addupdate(acc.at[bp, pl.ds(f * L, L)],
                                           accs[f])

                    keep = jnp.where(pred, 0.0, 1.0).astype(jnp.float32)
                    a2 = [a * keep for a in accs]
                    for h in range(GR // L):
                        for l in range(L):
                            wr = wvs[h][l]
                            for f in range(NFV):
                                a2[f] = a2[f] + chunk_v[rr + h * L + l,
                                                        pl.ds(f * L, L)] * wr
                    for f in range(NFV):
                        run_v[pl.ds(f * L, L)] = a2[f]

                @pl.when(b_first != b_last)
                def slow():
                    bpl = bp
                    a2 = list(accs)
                    for h in range(GR // L):
                        for l in range(L):
                            b = bvs[h][l]
                            wr = wvs[h][l]
                            pred = b != bpl

                            @pl.when(pred)
                            def flush(bpl=bpl, a2=list(a2)):
                                for f in range(NFV):
                                    plsc.addupdate(
                                        acc.at[bpl, pl.ds(f * L, L)], a2[f])

                            keep = jnp.where(pred, 0.0,
                                             1.0).astype(jnp.float32)
                            for f in range(NFV):
                                a2[f] = (a2[f] * keep
                                         + chunk_v[rr + h * L + l,
                                                   pl.ds(f * L, L)] * wr)
                            bpl = b
                    for f in range(NFV):
                        run_v[pl.ds(f * L, L)] = a2[f]

                return b_last

            bp_end = lax.fori_loop(0, CH // GR, group, b_lo)
            for f in range(NFV):
                sl = pl.ds(f * L, L)
                plsc.addupdate(acc.at[bp_end, sl], run_v[sl])

    maybe_start(0, bufs[0])

    def outer(o, carry):
        i0 = o * 2
        maybe_start(i0 + 1, bufs[1])
        process_chunk(i0, bufs[0])
        maybe_start(i0 + 2, bufs[0])
        process_chunk(i0 + 1, bufs[1])
        return carry

    lax.fori_loop(0, NMAX // 2, outer, 0)

    pltpu.sync_copy(acc.at[pl.ds(lo, HB)], shared.at[s])
    plsc.subcore_barrier()

    ro = s * ROWS_OUT
    pltpu.sync_copy(shared.at[0, pl.ds(ro, ROWS_OUT)], red)

    def redj(j, carry):
        pltpu.sync_copy(shared.at[j, pl.ds(ro, ROWS_OUT)], tmp)
        for r in range(ROWS_OUT):
            for f in range(NFV):
                sl = pl.ds(f * L, L)
                red[r, sl] = red[r, sl] + tmp[r, sl]
        return carry

    lax.fori_loop(1, NS, redj, 0)
    pltpu.sync_copy(red, out_hbm.at[pl.ds(lo + ro, ROWS_OUT)])


@jax.jit
def _pool(attr, w, bi):
    mesh = plsc.VectorSubcoreMesh(core_axis_name="c", subcore_axis_name="s",
                                  num_cores=NC, num_subcores=NS)
    return pl.kernel(
        _sc_body,
        out_type=jax.ShapeDtypeStruct((B, F), jnp.float32),
        mesh=mesh,
        scratch_types=[
            pltpu.VMEM((CH, F), jnp.float32),
            pltpu.VMEM((CH, F), jnp.float32),
            pltpu.VMEM((CH,), jnp.float32),
            pltpu.VMEM((CH,), jnp.float32),
            pltpu.VMEM((NMAX * CH,), jnp.int32),
            pltpu.VMEM((F,), jnp.float32),
            pltpu.VMEM((B, F), jnp.float32),
            pltpu.VMEM((ROWS_OUT, F), jnp.float32),
            pltpu.VMEM((ROWS_OUT, F), jnp.float32),
            pltpu.VMEM_SHARED((NS, HB, F), jnp.float32),
            pltpu.SemaphoreType.DMA,
            pltpu.SemaphoreType.DMA,
            pltpu.SemaphoreType.DMA,
        ],
    )(attr, w, bi)


def kernel(reference, attr, weights, batch_index):
    del reference
    return _pool(attr, weights.reshape(-1).astype(jnp.float32),
                 batch_index.astype(jnp.int32))


# speedup vs baseline: 1.2146x; 1.0678x over previous
import jax
import jax.numpy as jnp
from jax import lax
from jax.experimental import pallas as pl
from jax.experimental.pallas import tpu as pltpu
from jax.experimental.pallas import tpu_sc as plsc

N = 100000
F = 128
B = 256
NC = 2
NS = 16
L = 16

CH = 160
GR = 16
G = N // CH
NMAX = (G + NS - 1) // NS
HB = B // NC
ROWS_OUT = HB // NS
NFV = F // L


def _sc_body(attr_hbm, w_hbm, bi_hbm, out_hbm, chunk0, chunk1, wv0, wv1, bib,
             run_v, acc, red, idx_v, acc_sp, sem0, sem1, sem_bi):
    c = lax.axis_index("c")
    s = lax.axis_index("s")
    lo = c * HB

    zvec = jnp.zeros((L,), jnp.float32)
    n_my = (G - s + NS - 1) // NS

    def bi_start(i, carry):
        g = s + i * NS
        pltpu.async_copy(bi_hbm.at[pl.ds(g * CH, CH)],
                         bib.at[pl.ds(i * CH, CH)], sem_bi)
        return carry

    lax.fori_loop(0, n_my, bi_start, 0)

    def zrow(r, carry):
        for f in range(NFV):
            acc[lo + r, pl.ds(f * L, L)] = zvec
        return carry

    lax.fori_loop(0, HB, zrow, 0)

    def bi_drain(i, carry):
        pltpu.make_async_copy(bi_hbm.at[pl.ds(s * CH, CH)],
                              bib.at[pl.ds(i * CH, CH)], sem_bi).wait()
        return carry

    lax.fori_loop(0, n_my, bi_drain, 0)

    bufs = ((chunk0, wv0, sem0), (chunk1, wv1, sem1))

    def flags(i):
        off = i * CH
        b_lo = bib[pl.ds(off, L)][0]
        b_hi = bib[pl.ds(off + CH - L, L)][L - 1]
        proc = jnp.logical_and(jnp.logical_and(b_hi >= lo, b_lo < lo + HB),
                               i < n_my)
        return b_lo, proc

    def maybe_start(i, buf, proc):
        chunk_v, wv_v, sem = buf

        @pl.when(proc)
        def _():
            r0 = (s + i * NS) * CH
            pltpu.async_copy(attr_hbm.at[pl.ds(r0, CH)], chunk_v, sem)
            pltpu.async_copy(w_hbm.at[pl.ds(r0, CH)], wv_v, sem)

    def process_chunk(i, buf, b_lo, proc):
        chunk_v, wv_v, sem = buf

        @pl.when(proc)
        def _():
            r0 = (s + i * NS) * CH
            pltpu.make_async_copy(attr_hbm.at[pl.ds(r0, CH)], chunk_v,
                                  sem).wait()
            pltpu.make_async_copy(w_hbm.at[pl.ds(r0, CH)], wv_v, sem).wait()
            for f in range(NFV):
                run_v[pl.ds(f * L, L)] = zvec
            off = i * CH

            def group(r32, bp):
                rr = r32 * GR
                bvs = [bib[pl.ds(off + rr + h * L, L)] for h in range(GR // L)]
                wvs = [wv_v[pl.ds(rr + h * L, L)] for h in range(GR // L)]
                b_first = bvs[0][0]
                b_last = bvs[-1][L - 1]
                accs = [run_v[pl.ds(f * L, L)] for f in range(NFV)]

                @pl.when(b_first == b_last)
                def fast():
                    pred = b_first != bp

                    @pl.when(pred)
                    def flush():
                        for f in range(NFV):
                            plsc.addupdate(acc.at[bp, pl.ds(f * L, L)],
                                           accs[f])

                    keep = jnp.where(pred, 0.0, 1.0).astype(jnp.float32)
                    a2 = [a * keep for a in accs]
                    for h in range(GR // L):
                        for l in range(L):
                            wr = wvs[h][l]
                            for f in range(NFV):
                                a2[f] = a2[f] + chunk_v[rr + h * L + l,
                                                        pl.ds(f * L, L)] * wr
                    for f in range(NFV):
                        run_v[pl.ds(f * L, L)] = a2[f]

                @pl.when(b_first != b_last)
                def slow():
                    bpl = bp
                    a2 = list(accs)
                    for h in range(GR // L):
                        for l in range(L):
                            b = bvs[h][l]
                            wr = wvs[h][l]
                            pred = b != bpl

                            @pl.when(pred)
                            def flush(bpl=bpl, a2=list(a2)):
                                for f in range(NFV):
                                    plsc.addupdate(
                                        acc.at[bpl, pl.ds(f * L, L)], a2[f])

                            keep = jnp.where(pred, 0.0,
                                             1.0).astype(jnp.float32)
                            for f in range(NFV):
                                a2[f] = (a2[f] * keep
                                         + chunk_v[rr + h * L + l,
                                                   pl.ds(f * L, L)] * wr)
                            bpl = b
                    for f in range(NFV):
                        run_v[pl.ds(f * L, L)] = a2[f]

                return b_last

            bp_end = lax.fori_loop(0, CH // GR, group, b_lo)
            for f in range(NFV):
                sl = pl.ds(f * L, L)
                plsc.addupdate(acc.at[bp_end, sl], run_v[sl])

    ro = s * ROWS_OUT
    for r in range(ROWS_OUT):
        for f in range(NFV):
            red[r, pl.ds(f * L, L)] = zvec
    for j in range(HB // L):
        idx_v[pl.ds(j * L, L)] = lax.iota(jnp.int32, L) + j * L
    pltpu.sync_copy(red, acc_sp.at[pl.ds(ro, ROWS_OUT)])

    bl0, p0 = flags(0)
    maybe_start(0, bufs[0], p0)

    def outer(o, carry):
        i0 = o * 2
        bl_a, p_a = carry
        bl_b, p_b = flags(i0 + 1)
        maybe_start(i0 + 1, bufs[1], p_b)
        process_chunk(i0, bufs[0], bl_a, p_a)
        bl_c, p_c = flags(i0 + 2)
        maybe_start(i0 + 2, bufs[0], p_c)
        process_chunk(i0 + 1, bufs[1], bl_b, p_b)
        return (bl_c, p_c)

    lax.fori_loop(0, NMAX // 2, outer, (bl0, p0))

    plsc.subcore_barrier()
    pltpu.sync_copy(acc.at[pl.ds(lo, HB)], acc_sp.at[idx_v], add=True)
    plsc.subcore_barrier()
    pltpu.sync_copy(acc_sp.at[pl.ds(ro, ROWS_OUT)],
                    out_hbm.at[pl.ds(lo + ro, ROWS_OUT)])


@jax.jit
def _pool(attr, w, bi):
    mesh = plsc.VectorSubcoreMesh(core_axis_name="c", subcore_axis_name="s",
                                  num_cores=NC, num_subcores=NS)
    return pl.kernel(
        _sc_body,
        out_type=jax.ShapeDtypeStruct((B, F), jnp.float32),
        mesh=mesh,
        scratch_types=[
            pltpu.VMEM((CH, F), jnp.float32),
            pltpu.VMEM((CH, F), jnp.float32),
            pltpu.VMEM((CH,), jnp.float32),
            pltpu.VMEM((CH,), jnp.float32),
            pltpu.VMEM((NMAX * CH,), jnp.int32),
            pltpu.VMEM((F,), jnp.float32),
            pltpu.VMEM((B, F), jnp.float32),
            pltpu.VMEM((ROWS_OUT, F), jnp.float32),
            pltpu.VMEM((HB,), jnp.int32),
            pltpu.VMEM_SHARED((HB, F), jnp.float32),
            pltpu.SemaphoreType.DMA,
            pltpu.SemaphoreType.DMA,
            pltpu.SemaphoreType.DMA,
        ],
    )(attr, w, bi)


def kernel(reference, attr, weights, batch_index):
    del reference
    return _pool(attr, weights.reshape(-1).astype(jnp.float32),
                 batch_index.astype(jnp.int32))


# speedup vs baseline: 1.2280x; 1.0111x over previous
import jax
import jax.numpy as jnp
from jax import lax
from jax.experimental import pallas as pl
from jax.experimental.pallas import tpu as pltpu
from jax.experimental.pallas import tpu_sc as plsc

N = 100000
F = 128
B = 256
NC = 2
NS = 16
L = 16

CH = 160
GR = 16
G = N // CH
NMAX = (G + NS - 1) // NS
HB = B // NC
ROWS_OUT = HB // NS
NFV = F // L


def _sc_body(attr_hbm, w_hbm, bi_hbm, out_hbm, chunk0, chunk1, chunk2, wv0,
             wv1, wv2, bib, run_v, acc, red, idx_v, acc_sp, sem0, sem1, sem2,
             sem_bi):
    c = lax.axis_index("c")
    s = lax.axis_index("s")
    lo = c * HB

    zvec = jnp.zeros((L,), jnp.float32)
    n_my = (G - s + NS - 1) // NS

    def bi_start(i, carry):
        g = s + i * NS
        pltpu.async_copy(bi_hbm.at[pl.ds(g * CH, CH)],
                         bib.at[pl.ds(i * CH, CH)], sem_bi)
        return carry

    lax.fori_loop(0, n_my, bi_start, 0)

    def zrow(r, carry):
        for f in range(NFV):
            acc[lo + r, pl.ds(f * L, L)] = zvec
        return carry

    lax.fori_loop(0, HB, zrow, 0)

    ro = s * ROWS_OUT
    for r in range(ROWS_OUT):
        for f in range(NFV):
            red[r, pl.ds(f * L, L)] = zvec
    for j in range(HB // L):
        idx_v[pl.ds(j * L, L)] = lax.iota(jnp.int32, L) + j * L
    pltpu.sync_copy(red, acc_sp.at[pl.ds(ro, ROWS_OUT)])

    def bi_drain(i, carry):
        pltpu.make_async_copy(bi_hbm.at[pl.ds(s * CH, CH)],
                              bib.at[pl.ds(i * CH, CH)], sem_bi).wait()
        return carry

    lax.fori_loop(0, n_my, bi_drain, 0)

    bufs = ((chunk0, wv0, sem0), (chunk1, wv1, sem1), (chunk2, wv2, sem2))

    def flags(i):
        off = jnp.minimum(i, NMAX - 1) * CH
        b_lo = bib[pl.ds(off, L)][0]
        b_hi = bib[pl.ds(off + CH - L, L)][L - 1]
        proc = jnp.logical_and(jnp.logical_and(b_hi >= lo, b_lo < lo + HB),
                               i < n_my)
        return b_lo, proc

    def maybe_start(i, buf, proc):
        chunk_v, wv_v, sem = buf

        @pl.when(proc)
        def _():
            r0 = (s + i * NS) * CH
            pltpu.async_copy(attr_hbm.at[pl.ds(r0, CH)], chunk_v, sem)
            pltpu.async_copy(w_hbm.at[pl.ds(r0, CH)], wv_v, sem)

    def process_chunk(i, buf, b_lo, proc):
        chunk_v, wv_v, sem = buf

        @pl.when(proc)
        def _():
            r0 = (s + i * NS) * CH
            pltpu.make_async_copy(attr_hbm.at[pl.ds(r0, CH)], chunk_v,
                                  sem).wait()
            pltpu.make_async_copy(w_hbm.at[pl.ds(r0, CH)], wv_v, sem).wait()
            for f in range(NFV):
                run_v[pl.ds(f * L, L)] = zvec
            off = i * CH

            def group(r32, bp):
                rr = r32 * GR
                bvs = [bib[pl.ds(off + rr + h * L, L)] for h in range(GR // L)]
                wvs = [wv_v[pl.ds(rr + h * L, L)] for h in range(GR // L)]
                b_first = bvs[0][0]
                b_last = bvs[-1][L - 1]
                accs = [run_v[pl.ds(f * L, L)] for f in range(NFV)]

                @pl.when(b_first == b_last)
                def fast():
                    pred = b_first != bp

                    @pl.when(pred)
                    def flush():
                        for f in range(NFV):
                            plsc.addupdate(acc.at[bp, pl.ds(f * L, L)],
                                           accs[f])

                    keep = jnp.where(pred, 0.0, 1.0).astype(jnp.float32)
                    a2 = [a * keep for a in accs]
                    for h in range(GR // L):
                        for l in range(L):
                            wr = wvs[h][l]
                            for f in range(NFV):
                                a2[f] = a2[f] + chunk_v[rr + h * L + l,
                                                        pl.ds(f * L, L)] * wr
                    for f in range(NFV):
                        run_v[pl.ds(f * L, L)] = a2[f]

                @pl.when(b_first != b_last)
                def slow():
                    bpl = bp
                    a2 = list(accs)
                    for h in range(GR // L):
                        for l in range(L):
                            b = bvs[h][l]
                            wr = wvs[h][l]
                            pred = b != bpl

                            @pl.when(pred)
                            def flush(bpl=bpl, a2=list(a2)):
                                for f in range(NFV):
                                    plsc.addupdate(
                                        acc.at[bpl, pl.ds(f * L, L)], a2[f])

                            keep = jnp.where(pred, 0.0,
                                             1.0).astype(jnp.float32)
                            for f in range(NFV):
                                a2[f] = (a2[f] * keep
                                         + chunk_v[rr + h * L + l,
                                                   pl.ds(f * L, L)] * wr)
                            bpl = b
                    for f in range(NFV):
                        run_v[pl.ds(f * L, L)] = a2[f]

                return b_last

            bp_end = lax.fori_loop(0, CH // GR, group, b_lo)
            for f in range(NFV):
                sl = pl.ds(f * L, L)
                plsc.addupdate(acc.at[bp_end, sl], run_v[sl])

    f0 = flags(0)
    f1 = flags(1)
    maybe_start(0, bufs[0], f0[1])
    maybe_start(1, bufs[1], f1[1])

    def outer(o, carry):
        i0 = o * 3
        fa, fb = carry[:2], carry[2:]
        fc = flags(i0 + 2)
        maybe_start(i0 + 2, bufs[2], fc[1])
        process_chunk(i0, bufs[0], fa[0], fa[1])
        fd = flags(i0 + 3)
        maybe_start(i0 + 3, bufs[0], fd[1])
        process_chunk(i0 + 1, bufs[1], fb[0], fb[1])
        fe = flags(i0 + 4)
        maybe_start(i0 + 4, bufs[1], fe[1])
        process_chunk(i0 + 2, bufs[2], fc[0], fc[1])
        return (*fd, *fe)

    lax.fori_loop(0, (NMAX + 2) // 3, outer, (*f0, *f1))

    plsc.subcore_barrier()
    pltpu.sync_copy(acc.at[pl.ds(lo, HB)], acc_sp.at[idx_v], add=True)
    plsc.subcore_barrier()
    pltpu.sync_copy(acc_sp.at[pl.ds(ro, ROWS_OUT)],
                    out_hbm.at[pl.ds(lo + ro, ROWS_OUT)])


@jax.jit
def _pool(attr, w, bi):
    mesh = plsc.VectorSubcoreMesh(core_axis_name="c", subcore_axis_name="s",
                                  num_cores=NC, num_subcores=NS)
    return pl.kernel(
        _sc_body,
        out_type=jax.ShapeDtypeStruct((B, F), jnp.float32),
        mesh=mesh,
        scratch_types=[
            pltpu.VMEM((CH, F), jnp.float32),
            pltpu.VMEM((CH, F), jnp.float32),
            pltpu.VMEM((CH, F), jnp.float32),
            pltpu.VMEM((CH,), jnp.float32),
            pltpu.VMEM((CH,), jnp.float32),
            pltpu.VMEM((CH,), jnp.float32),
            pltpu.VMEM((NMAX * CH,), jnp.int32),
            pltpu.VMEM((F,), jnp.float32),
            pltpu.VMEM((B, F), jnp.float32),
            pltpu.VMEM((ROWS_OUT, F), jnp.float32),
            pltpu.VMEM((HB,), jnp.int32),
            pltpu.VMEM_SHARED((HB, F), jnp.float32),
            pltpu.SemaphoreType.DMA,
            pltpu.SemaphoreType.DMA,
            pltpu.SemaphoreType.DMA,
            pltpu.SemaphoreType.DMA,
        ],
    )(attr, w, bi)


def kernel(reference, attr, weights, batch_index):
    del reference
    return _pool(attr, weights.reshape(-1).astype(jnp.float32),
                 batch_index.astype(jnp.int32))


# speedup vs baseline: 1.2419x; 1.0113x over previous
import jax
import jax.numpy as jnp
from jax import lax
from jax.experimental import pallas as pl
from jax.experimental.pallas import tpu as pltpu
from jax.experimental.pallas import tpu_sc as plsc

N = 100000
F = 128
B = 256
NC = 2
NS = 16
L = 16

CH = 400
GR = 16
PAD = 8
G = N // CH
NMAX = (G + NS - 1) // NS
HB = B // NC
ROWS_OUT = HB // NS
NFV = F // L


def _sc_body(attr_hbm, w_hbm, bi_hbm, out_hbm, chunk0, chunk1, wv0, wv1, bib,
             run_v, acc, red, idx_v, acc_sp, sem0, sem1, sem_bi):
    c = lax.axis_index("c")
    s = lax.axis_index("s")
    lo = c * HB

    def arow(b):
        return jnp.clip(b - lo, -PAD, HB) + PAD

    zvec = jnp.zeros((L,), jnp.float32)
    n_my = (G - s + NS - 1) // NS

    def bi_start(i, carry):
        g = s + i * NS
        pltpu.async_copy(bi_hbm.at[pl.ds(g * CH, CH)],
                         bib.at[pl.ds(i * CH, CH)], sem_bi)
        return carry

    lax.fori_loop(0, n_my, bi_start, 0)

    def zrow(r, carry):
        for f in range(NFV):
            acc[PAD + r, pl.ds(f * L, L)] = zvec
        return carry

    lax.fori_loop(0, HB, zrow, 0)

    ro = s * ROWS_OUT
    for r in range(ROWS_OUT):
        for f in range(NFV):
            red[r, pl.ds(f * L, L)] = zvec
    for j in range(HB // L):
        idx_v[pl.ds(j * L, L)] = lax.iota(jnp.int32, L) + j * L
    pltpu.sync_copy(red, acc_sp.at[pl.ds(ro, ROWS_OUT)])

    def bi_drain(i, carry):
        pltpu.make_async_copy(bi_hbm.at[pl.ds(s * CH, CH)],
                              bib.at[pl.ds(i * CH, CH)], sem_bi).wait()
        return carry

    lax.fori_loop(0, n_my, bi_drain, 0)

    bufs = ((chunk0, wv0, sem0), (chunk1, wv1, sem1))

    def flags(i):
        off = jnp.minimum(i, NMAX - 1) * CH
        b_lo = bib[pl.ds(off, L)][0]
        b_hi = bib[pl.ds(off + CH - L, L)][L - 1]
        proc = jnp.logical_and(jnp.logical_and(b_hi >= lo, b_lo < lo + HB),
                               i < n_my)
        return b_lo, proc

    def maybe_start(i, buf, proc):
        chunk_v, wv_v, sem = buf

        @pl.when(proc)
        def _():
            r0 = (s + i * NS) * CH
            pltpu.async_copy(attr_hbm.at[pl.ds(r0, CH)], chunk_v, sem)
            pltpu.async_copy(w_hbm.at[pl.ds(r0, CH)], wv_v, sem)

    def process_chunk(i, buf, b_lo, proc):
        chunk_v, wv_v, sem = buf

        @pl.when(proc)
        def _():
            r0 = (s + i * NS) * CH
            pltpu.make_async_copy(attr_hbm.at[pl.ds(r0, CH)], chunk_v,
                                  sem).wait()
            pltpu.make_async_copy(w_hbm.at[pl.ds(r0, CH)], wv_v, sem).wait()
            for f in range(NFV):
                run_v[pl.ds(f * L, L)] = zvec
            off = i * CH

            def group(r32, bp):
                rr = r32 * GR
                bvs = [bib[pl.ds(off + rr + h * L, L)] for h in range(GR // L)]
                wvs = [wv_v[pl.ds(rr + h * L, L)] for h in range(GR // L)]
                b_first = bvs[0][0]
                b_last = bvs[-1][L - 1]
                accs = [run_v[pl.ds(f * L, L)] for f in range(NFV)]

                @pl.when(b_first == b_last)
                def fast():
                    pred = b_first != bp

                    bpr = arow(bp)

                    @pl.when(pred)
                    def flush():
                        for f in range(NFV):
                            plsc.addupdate(acc.at[bpr, pl.ds(f * L, L)],
                                           accs[f])

                    keep = jnp.where(pred, 0.0, 1.0).astype(jnp.float32)
                    a2 = [a * keep for a in accs]
                    for h in range(GR // L):
                        for l in range(L):
                            wr = wvs[h][l]
                            for f in range(NFV):
                                a2[f] = a2[f] + chunk_v[rr + h * L + l,
                                                        pl.ds(f * L, L)] * wr
                    for f in range(NFV):
                        run_v[pl.ds(f * L, L)] = a2[f]

                @pl.when(b_first != b_last)
                def slow():
                    bpl = bp
                    a2 = list(accs)
                    for h in range(GR // L):
                        for l in range(L):
                            b = bvs[h][l]
                            wr = wvs[h][l]
                            pred = b != bpl

                            bpr = arow(bpl)

                            @pl.when(pred)
                            def flush(bpr=bpr, a2=list(a2)):
                                for f in range(NFV):
                                    plsc.addupdate(
                                        acc.at[bpr, pl.ds(f * L, L)], a2[f])

                            keep = jnp.where(pred, 0.0,
                                             1.0).astype(jnp.float32)
                            for f in range(NFV):
                                a2[f] = (a2[f] * keep
                                         + chunk_v[rr + h * L + l,
                                                   pl.ds(f * L, L)] * wr)
                            bpl = b
                    for f in range(NFV):
                        run_v[pl.ds(f * L, L)] = a2[f]

                return b_last

            bp_end = arow(lax.fori_loop(0, CH // GR, group, b_lo))
            for f in range(NFV):
                sl = pl.ds(f * L, L)
                plsc.addupdate(acc.at[bp_end, sl], run_v[sl])

    f0 = flags(0)
    maybe_start(0, bufs[0], f0[1])

    def outer(o, carry):
        i0 = o * 2
        bl_a, p_a = carry
        bl_b, p_b = flags(i0 + 1)
        maybe_start(i0 + 1, bufs[1], p_b)
        process_chunk(i0, bufs[0], bl_a, p_a)
        bl_c, p_c = flags(i0 + 2)
        maybe_start(i0 + 2, bufs[0], p_c)
        process_chunk(i0 + 1, bufs[1], bl_b, p_b)
        return (bl_c, p_c)

    lax.fori_loop(0, (NMAX + 1) // 2, outer, f0)

    plsc.subcore_barrier()
    pltpu.sync_copy(acc.at[pl.ds(PAD, HB)], acc_sp.at[idx_v], add=True)
    plsc.subcore_barrier()
    pltpu.sync_copy(acc_sp.at[pl.ds(ro, ROWS_OUT)],
                    out_hbm.at[pl.ds(lo + ro, ROWS_OUT)])


@jax.jit
def _pool(attr, w, bi):
    mesh = plsc.VectorSubcoreMesh(core_axis_name="c", subcore_axis_name="s",
                                  num_cores=NC, num_subcores=NS)
    return pl.kernel(
        _sc_body,
        out_type=jax.ShapeDtypeStruct((B, F), jnp.float32),
        mesh=mesh,
        scratch_types=[
            pltpu.VMEM((CH, F), jnp.float32),
            pltpu.VMEM((CH, F), jnp.float32),
            pltpu.VMEM((CH,), jnp.float32),
            pltpu.VMEM((CH,), jnp.float32),
            pltpu.VMEM((NMAX * CH,), jnp.int32),
            pltpu.VMEM((F,), jnp.float32),
            pltpu.VMEM((HB + 2 * PAD, F), jnp.float32),
            pltpu.VMEM((ROWS_OUT, F), jnp.float32),
            pltpu.VMEM((HB,), jnp.int32),
            pltpu.VMEM_SHARED((HB, F), jnp.float32),
            pltpu.SemaphoreType.DMA,
            pltpu.SemaphoreType.DMA,
            pltpu.SemaphoreType.DMA,
        ],
    )(attr, w, bi)


def kernel(reference, attr, weights, batch_index):
    del reference
    return _pool(attr, weights.reshape(-1).astype(jnp.float32),
                 batch_index.astype(jnp.int32))
